# speedup vs baseline: 1.8996x; 1.8996x over previous
import functools

import jax
import jax.numpy as jnp
from jax import lax
from jax.experimental import pallas as pl
from jax.experimental.pallas import tpu as pltpu
from jax.experimental.pallas import tpu_sc as plsc

N = 10000
E = 320000
EMB = 128
DE = 16
NG = 64
OUT = 64
NL = 5

N_PAD = 10240
NC = 2
NS = 16
LANES = 16
EDGES_PER_TILE = E // (NC * NS)
CHUNK = 80
NCHUNKS = EDGES_PER_TILE // CHUNK
ROWS_PER_TILE = N_PAD // NS
ZROWS = 128


def _sc_scatter_body(x_hbm, e_hbm, src_hbm, dst_hbm, out_hbm,
                     src_v, dst_v, xrow_v, m_v, zero_v, agg_s, sem):
  c = lax.axis_index("c")
  s = lax.axis_index("s")
  tile = c * NS + s

  zvec = jnp.zeros((LANES,), jnp.float32)

  def zrow(r, _):
    for j in range(EMB // LANES):
      zero_v[r, pl.ds(j * LANES, LANES)] = zvec
    return 0

  lax.fori_loop(0, ZROWS, zrow, 0, unroll=4)
  row0 = s * ROWS_PER_TILE
  for k in range(ROWS_PER_TILE // ZROWS):
    pltpu.sync_copy(zero_v, agg_s.at[pl.ds(row0 + k * ZROWS, ZROWS)])
  plsc.subcore_barrier()

  ebase = tile * EDGES_PER_TILE

  def chunk(jc, _):
    off = pl.multiple_of(ebase + jc * CHUNK, 8)
    pltpu.sync_copy(src_hbm.at[pl.ds(off, CHUNK)], src_v)
    pltpu.sync_copy(dst_hbm.at[pl.ds(off, CHUNK)], dst_v)
    gat = pltpu.async_copy(x_hbm.at[src_v], xrow_v, sem)
    pltpu.sync_copy(e_hbm.at[pl.ds(off, CHUNK)], m_v)
    gat.wait()

    def row(r, _):
      for j in range(EMB // LANES):
        sl = pl.ds(j * LANES, LANES)
        m_v[r, sl] = jnp.maximum(m_v[r, sl] + xrow_v[r, sl], 0.0)
      return 0

    lax.fori_loop(0, CHUNK, row, 0, unroll=2)
    pltpu.sync_copy(m_v, agg_s.at[dst_v], add=True)
    return 0

  lax.fori_loop(0, NCHUNKS, chunk, 0)
  plsc.subcore_barrier()

  pltpu.sync_copy(agg_s.at[pl.ds(row0, ROWS_PER_TILE)],
                  out_hbm.at[c, pl.ds(row0, ROWS_PER_TILE)])


_sc_scatter = pl.kernel(
    _sc_scatter_body,
    out_type=jax.ShapeDtypeStruct((NC, N_PAD, EMB), jnp.float32),
    mesh=plsc.VectorSubcoreMesh(core_axis_name="c", subcore_axis_name="s"),
    scratch_types=[
        pltpu.VMEM((CHUNK,), jnp.int32),
        pltpu.VMEM((CHUNK,), jnp.int32),
        pltpu.VMEM((CHUNK, EMB), jnp.float32),
        pltpu.VMEM((CHUNK, EMB), jnp.float32),
        pltpu.VMEM((ZROWS, EMB), jnp.float32),
        pltpu.VMEM_SHARED((N_PAD, EMB), jnp.float32),
        pltpu.SemaphoreType.DMA,
    ],
)


_EBLK = 8000


def _edge_enc_body(a_ref, w_ref, b_ref, o_ref):
  o_ref[...] = (
      jnp.dot(a_ref[...], w_ref[...], preferred_element_type=jnp.float32)
      + b_ref[...])


_edge_enc = pl.pallas_call(
    _edge_enc_body,
    grid=(E // _EBLK,),
    in_specs=[
        pl.BlockSpec((_EBLK, DE), lambda i: (i, 0)),
        pl.BlockSpec((DE, EMB), lambda i: (0, 0)),
        pl.BlockSpec((1, EMB), lambda i: (0, 0)),
    ],
    out_specs=pl.BlockSpec((_EBLK, EMB), lambda i: (i, 0)),
    out_shape=jax.ShapeDtypeStruct((E, EMB), jnp.float32),
)

_NBLK = 2048


def _mlp_body(eps_ref, x_ref, agg_ref, w1_ref, b1_ref, w2_ref, b2_ref, o_ref):
  t = (1.0 + eps_ref[0]) * x_ref[...] + agg_ref[0] + agg_ref[1]
  u = jnp.dot(t, w1_ref[...], preferred_element_type=jnp.float32) + b1_ref[...]
  u = jnp.maximum(u, 0.0)
  o_ref[...] = (
      jnp.dot(u, w2_ref[...], preferred_element_type=jnp.float32)
      + b2_ref[...])


_node_mlp = pl.pallas_call(
    _mlp_body,
    grid=(N_PAD // _NBLK,),
    in_specs=[
        pl.BlockSpec(memory_space=pltpu.SMEM),
        pl.BlockSpec((_NBLK, EMB), lambda i: (i, 0)),
        pl.BlockSpec((NC, _NBLK, EMB), lambda i: (0, i, 0)),
        pl.BlockSpec((EMB, 2 * EMB), lambda i: (0, 0)),
        pl.BlockSpec((1, 2 * EMB), lambda i: (0, 0)),
        pl.BlockSpec((2 * EMB, EMB), lambda i: (0, 0)),
        pl.BlockSpec((1, EMB), lambda i: (0, 0)),
    ],
    out_specs=pl.BlockSpec((_NBLK, EMB), lambda i: (i, 0)),
    out_shape=jax.ShapeDtypeStruct((N_PAD, EMB), jnp.float32),
)


def _pool_body(h_ref, batch_ref, lw1_ref, lb1_ref, lw2_ref, lb2_ref, o_ref,
               acc_ref):
  i = pl.program_id(0)

  @pl.when(i == 0)
  def _():
    acc_ref[...] = jnp.zeros_like(acc_ref)

  gids = lax.broadcasted_iota(jnp.int32, (NG, _NBLK), 0)
  onehot = (gids == batch_ref[0, 0, :][None, :]).astype(jnp.float32)
  acc_ref[...] += jnp.dot(onehot, h_ref[...],
                          preferred_element_type=jnp.float32)

  @pl.when(i == pl.num_programs(0) - 1)
  def _():
    u = jnp.dot(acc_ref[...], lw1_ref[...],
                preferred_element_type=jnp.float32) + lb1_ref[...]
    u = jnp.maximum(u, 0.0)
    o_ref[...] = (
        jnp.dot(u, lw2_ref[...], preferred_element_type=jnp.float32)
        + lb2_ref[...])


_pool_head = pl.pallas_call(
    _pool_body,
    grid=(N_PAD // _NBLK,),
    in_specs=[
        pl.BlockSpec((_NBLK, EMB), lambda i: (i, 0)),
        pl.BlockSpec((1, 1, _NBLK), lambda i: (i, 0, 0)),
        pl.BlockSpec((EMB, EMB // 2), lambda i: (0, 0)),
        pl.BlockSpec((1, EMB // 2), lambda i: (0, 0)),
        pl.BlockSpec((EMB // 2, OUT), lambda i: (0, 0)),
        pl.BlockSpec((1, OUT), lambda i: (0, 0)),
    ],
    out_specs=pl.BlockSpec((NG, OUT), lambda i: (0, 0)),
    out_shape=jax.ShapeDtypeStruct((NG, OUT), jnp.float32),
    scratch_shapes=[pltpu.VMEM((NG, EMB), jnp.float32)],
)


@jax.jit
def kernel(x, edge_index, edge_attr, batch, eW, eb, eps, W1, b1, bn_g, bn_b,
           W2, b2, lW1, lb1, lW2, lb2):
  src = edge_index[0].astype(jnp.int32)
  dst = edge_index[1].astype(jnp.int32)
  batch32 = batch.astype(jnp.int32)

  h = jnp.pad(x, ((0, N_PAD - N), (0, 0)))
  batch_pad = jnp.concatenate(
      [batch32, jnp.full((N_PAD - N,), NG, jnp.int32)])
  batch3 = batch_pad.reshape(N_PAD // _NBLK, 1, _NBLK)

  bn_scale = bn_g / jnp.sqrt(1.0 + 1e-5)
  W1eff = W1 * bn_scale[:, None, :]
  b1eff = b1 * bn_scale + bn_b

  for i in range(NL):
    e = _edge_enc(edge_attr, eW[i], eb[i].reshape(1, EMB))
    agg = _sc_scatter(h, e, src, dst)
    h = _node_mlp(eps[i].reshape(1), h, agg, W1eff[i],
                  b1eff[i].reshape(1, 2 * EMB), W2[i],
                  b2[i].reshape(1, EMB))

  return _pool_head(h, batch3, lW1, lb1.reshape(1, EMB // 2), lW2,
                    lb2.reshape(1, OUT))
